# baseline (speedup 1.0000x reference)
import jax
import jax.numpy as jnp
from jax import lax
from jax.experimental import pallas as pl
from jax.experimental.pallas import tpu as pltpu
from jax.experimental.pallas import tpu_sc as plsc

D = 128
KN = 32
SN = 8
BS = 1024
LANES = 16
DC = D // LANES
ROW_E = KN * SN * D
ROW_T = KN * D
ROW_M = KN * SN
NW = 32
BPW = BS // NW


def _sc_body(self_hbm, ent_hbm, edge_hbm, mask_hbm, sv_hbm, nv_hbm,
             ebuf0, ebuf1, tbuf0, tbuf1, mbuf, sbuf, svbuf, nvbuf,
             se0, se1, st0, st1, spro, snv):
  wid = lax.axis_index("s") * 2 + lax.axis_index("c")
  base = wid * BPW

  cm = pltpu.async_copy(mask_hbm.at[pl.ds(base * ROW_M, BPW * ROW_M)], mbuf, spro)
  cs = pltpu.async_copy(self_hbm.at[pl.ds(base * D, BPW * D)], sbuf, spro)
  pltpu.async_copy(edge_hbm.at[pl.ds(base * ROW_E, ROW_E)], ebuf0, se0)
  pltpu.async_copy(ent_hbm.at[pl.ds(base * ROW_T, ROW_T)], tbuf0, st0)
  pltpu.async_copy(edge_hbm.at[pl.ds((base + 1) * ROW_E, ROW_E)], ebuf1, se1)
  pltpu.async_copy(ent_hbm.at[pl.ds((base + 1) * ROW_T, ROW_T)], tbuf1, st1)
  cm.wait()
  cs.wait()

  def do_row(i_local, ebuf, tbuf, sem_e, sem_t):
    pltpu.make_async_copy(edge_hbm.at[pl.ds(0, ROW_E)], ebuf, sem_e).wait()
    pltpu.make_async_copy(ent_hbm.at[pl.ds(0, ROW_T)], tbuf, sem_t).wait()

    @pl.when(i_local > 0)
    def _():
      pltpu.make_async_copy(nvbuf, nv_hbm.at[pl.ds(0, ROW_T)], snv).wait()

    mrow = i_local * ROW_M

    def k2body(k2, sacc):
      mvec = mbuf[pl.ds(pl.multiple_of(mrow + k2 * 2 * SN, LANES), LANES)]
      out = list(sacc)
      for par in range(2):
        k = k2 * 2 + par
        mv = [jnp.full((LANES,), mvec[par * SN + s_]) for s_ in range(SN)]
        cnt = mv[0]
        for s_ in range(1, SN):
          cnt = cnt + mv[s_]
        scale = 0.5 / jnp.maximum(cnt, 1.0)
        ke = k * (SN * D)
        kt = k * D
        for c in range(DC):
          acc = mv[0] * ebuf[pl.ds(pl.multiple_of(ke + c * LANES, LANES), LANES)]
          for s_ in range(1, SN):
            acc = acc + mv[s_] * ebuf[
                pl.ds(pl.multiple_of(ke + s_ * D + c * LANES, LANES), LANES)]
          off = pl.multiple_of(kt + c * LANES, LANES)
          nvv = tbuf[pl.ds(off, LANES)] + acc * scale
          nvbuf[pl.ds(off, LANES)] = nvv
          out[c] = out[c] + nvv
      return tuple(out)

    zero = jnp.zeros((LANES,), jnp.float32)
    sacc = lax.fori_loop(0, KN // 2, k2body, (zero,) * DC)

    for c in range(DC):
      off = pl.multiple_of(i_local * D + c * LANES, LANES)
      svbuf[pl.ds(off, LANES)] = sbuf[pl.ds(off, LANES)] + sacc[c] * (1.0 / (2.0 * KN))

    @pl.when(i_local < BPW - 2)
    def _():
      r = base + i_local + 2
      pltpu.async_copy(edge_hbm.at[pl.ds(r * ROW_E, ROW_E)], ebuf, sem_e)
      pltpu.async_copy(ent_hbm.at[pl.ds(r * ROW_T, ROW_T)], tbuf, sem_t)

    pltpu.async_copy(nvbuf, nv_hbm.at[pl.ds((base + i_local) * ROW_T, ROW_T)], snv)

  def loop_body(i2, carry):
    do_row(2 * i2, ebuf0, tbuf0, se0, st0)
    do_row(2 * i2 + 1, ebuf1, tbuf1, se1, st1)
    return carry

  lax.fori_loop(0, BPW // 2, loop_body, 0)
  pltpu.make_async_copy(nvbuf, nv_hbm.at[pl.ds(0, ROW_T)], snv).wait()
  pltpu.sync_copy(svbuf, sv_hbm.at[pl.ds(base * D, BPW * D)])


_mesh = plsc.VectorSubcoreMesh(core_axis_name="c", subcore_axis_name="s")

_kern = pl.kernel(
    _sc_body,
    out_type=[
        jax.ShapeDtypeStruct((BS * D,), jnp.float32),
        jax.ShapeDtypeStruct((BS * ROW_T,), jnp.float32),
    ],
    mesh=_mesh,
    scratch_types=[
        pltpu.VMEM((ROW_E,), jnp.float32),
        pltpu.VMEM((ROW_E,), jnp.float32),
        pltpu.VMEM((ROW_T,), jnp.float32),
        pltpu.VMEM((ROW_T,), jnp.float32),
        pltpu.VMEM((BPW * ROW_M,), jnp.float32),
        pltpu.VMEM((BPW * D,), jnp.float32),
        pltpu.VMEM((BPW * D,), jnp.float32),
        pltpu.VMEM((ROW_T,), jnp.float32),
        pltpu.SemaphoreType.DMA,
        pltpu.SemaphoreType.DMA,
        pltpu.SemaphoreType.DMA,
        pltpu.SemaphoreType.DMA,
        pltpu.SemaphoreType.DMA,
        pltpu.SemaphoreType.DMA,
    ],
)


def kernel(self_vectors, neighbor_entity_vectors, neighbor_edge_vectors, masks, W, b):
  sf = self_vectors.reshape(-1)
  tf = neighbor_entity_vectors.reshape(-1)
  ef = neighbor_edge_vectors.reshape(-1)
  mf = masks.reshape(-1)
  sv_flat, nv_flat = _kern(sf, tf, ef, mf)
  return (sv_flat.reshape(BS, 1, D), nv_flat.reshape(BS, 1, KN, D))
